# baseline (speedup 1.0000x reference)
import functools

import jax
import jax.numpy as jnp
import numpy as np
from jax import lax
from jax.experimental import pallas as pl
from jax.experimental.pallas import tpu as pltpu
from jax.experimental.pallas import tpu_sc as plsc

_VOCAB = 100000
_D = 128
_SEQ = 2048
_BATCH = 4
_NC = 2
_NS = 16
_NW = _NC * _NS
_ROWS = (_BATCH * _SEQ) // _NW


def _pe_table() -> np.ndarray:
    pos = np.arange(_SEQ, dtype=np.float32)[:, None]
    div = np.exp(np.arange(0, _D, 2, dtype=np.float32) * (-np.log(10000.0) / _D))
    pe = np.zeros((_SEQ, _D), dtype=np.float32)
    pe[:, 0::2] = np.sin(pos * div)
    pe[:, 1::2] = np.cos(pos * div)
    return pe


_PE = _pe_table()


_G = 2
_C = _ROWS // _G


def _sc_body(x_hbm, pe_hbm, table_hbm, out_hbm,
             idx_v, gb0, gb1, p0, p1, acc,
             sp, sg0, sg1, sa0, sa1, so0, so1):
    s_idx = lax.axis_index("s")
    wid = s_idx * _NC + lax.axis_index("c")
    base = wid * _ROWS
    batch = wid // (_SEQ // _ROWS)
    col = lax.rem(base, _SEQ)
    region = s_idx * _ROWS
    gbufs = (gb0, gb1)
    pbufs = (p0, p1)
    sgs = (sg0, sg1)
    sas = (sa0, sa1)
    sos = (so0, so1)
    pe_load = pltpu.async_copy(
        pe_hbm.at[pl.ds(col, _ROWS)], acc.at[pl.ds(region, _ROWS)], sp)
    pltpu.sync_copy(x_hbm.at[batch, pl.ds(col, _ROWS)], idx_v)
    gathers = [
        pltpu.async_copy(
            table_hbm.at[idx_v.at[pl.ds(g * _C, _C)]], gbufs[g], sgs[g])
        for g in range(_G)
    ]
    for g in range(_G):
        for k in range(_C // 16):
            pbufs[g][pl.ds(k * 16, 16)] = (
                region + g * _C + k * 16 + lax.iota(jnp.int32, 16))
    pe_load.wait()
    adds = []
    for g in range(_G):
        gathers[g].wait()
        adds.append(pltpu.async_copy(
            gbufs[g], acc.at[pbufs[g]], sas[g], add=True))
    outs = []
    for g in range(_G):
        adds[g].wait()
        outs.append(pltpu.async_copy(
            acc.at[pl.ds(region + g * _C, _C)],
            out_hbm.at[pl.ds(base + g * _C, _C)], sos[g]))
    for o in outs:
        o.wait()


@functools.partial(jax.jit, static_argnames=())
def _run(x2d, pe, table):
    mesh = plsc.VectorSubcoreMesh(core_axis_name="c", subcore_axis_name="s")
    f = pl.kernel(
        _sc_body,
        mesh=mesh,
        out_type=jax.ShapeDtypeStruct((_BATCH * _SEQ, _D), jnp.float32),
        scratch_types=(
            [pltpu.VMEM((_ROWS,), jnp.int32)]
            + [pltpu.VMEM((_C, _D), jnp.float32)] * _G
            + [pltpu.VMEM((_C,), jnp.int32)] * _G
            + [pltpu.VMEM_SHARED((_NS * _ROWS, _D), jnp.float32)]
            + [pltpu.SemaphoreType.DMA] * (1 + 3 * _G)
        ),
    )
    return f(x2d, pe, table)


def kernel(x, table):
    out = _run(x, _PE, table)
    return out.reshape(_BATCH, _SEQ, _D)


# speedup vs baseline: 1.0492x; 1.0492x over previous
import functools

import jax
import jax.numpy as jnp
import numpy as np
from jax import lax
from jax.experimental import pallas as pl
from jax.experimental.pallas import tpu as pltpu
from jax.experimental.pallas import tpu_sc as plsc

_VOCAB = 100000
_D = 128
_SEQ = 2048
_BATCH = 4
_NC = 2
_NS = 16
_NW = _NC * _NS
_ROWS = (_BATCH * _SEQ) // _NW


def _pe_table() -> np.ndarray:
    pos = np.arange(_SEQ, dtype=np.float32)[:, None]
    div = np.exp(np.arange(0, _D, 2, dtype=np.float32) * (-np.log(10000.0) / _D))
    pe = np.zeros((_SEQ, _D), dtype=np.float32)
    pe[:, 0::2] = np.sin(pos * div)
    pe[:, 1::2] = np.cos(pos * div)
    return pe


_PE = _pe_table()


_G = 8
_C = _ROWS // _G


def _sc_body(x_hbm, pe_hbm, table_hbm, out_hbm, idx_v, *rest):
    gbufs = rest[:_G]
    pbufs = rest[_G:2 * _G]
    acc = rest[2 * _G]
    sp = rest[2 * _G + 1]
    sgs = rest[2 * _G + 2:2 * _G + 2 + _G]
    sas = rest[2 * _G + 2 + _G:2 * _G + 2 + 2 * _G]
    sos = rest[2 * _G + 2 + 2 * _G:]
    s_idx = lax.axis_index("s")
    wid = s_idx * _NC + lax.axis_index("c")
    base = wid * _ROWS
    batch = wid // (_SEQ // _ROWS)
    col = lax.rem(base, _SEQ)
    region = s_idx * _ROWS
    pe_load = pltpu.async_copy(
        pe_hbm.at[pl.ds(col, _ROWS)], acc.at[pl.ds(region, _ROWS)], sp)
    pltpu.sync_copy(x_hbm.at[batch, pl.ds(col, _ROWS)], idx_v)
    gathers = [
        pltpu.async_copy(
            table_hbm.at[idx_v.at[pl.ds(g * _C, _C)]], gbufs[g], sgs[g])
        for g in range(_G)
    ]
    for g in range(_G):
        for k in range(_C // 16):
            pbufs[g][pl.ds(k * 16, 16)] = (
                region + g * _C + k * 16 + lax.iota(jnp.int32, 16))
    pe_load.wait()
    adds = []
    for g in range(_G):
        gathers[g].wait()
        adds.append(pltpu.async_copy(
            gbufs[g], acc.at[pbufs[g]], sas[g], add=True))
    outs = []
    for g in range(_G):
        adds[g].wait()
        outs.append(pltpu.async_copy(
            acc.at[pl.ds(region + g * _C, _C)],
            out_hbm.at[pl.ds(base + g * _C, _C)], sos[g]))
    for o in outs:
        o.wait()


@functools.partial(jax.jit, static_argnames=())
def _run(x2d, pe, table):
    mesh = plsc.VectorSubcoreMesh(core_axis_name="c", subcore_axis_name="s")
    f = pl.kernel(
        _sc_body,
        mesh=mesh,
        out_type=jax.ShapeDtypeStruct((_BATCH * _SEQ, _D), jnp.float32),
        scratch_types=(
            [pltpu.VMEM((_ROWS,), jnp.int32)]
            + [pltpu.VMEM((_C, _D), jnp.float32)] * _G
            + [pltpu.VMEM((_C,), jnp.int32)] * _G
            + [pltpu.VMEM_SHARED((_NS * _ROWS, _D), jnp.float32)]
            + [pltpu.SemaphoreType.DMA] * (1 + 3 * _G)
        ),
    )
    return f(x2d, pe, table)


def kernel(x, table):
    out = _run(x, _PE, table)
    return out.reshape(_BATCH, _SEQ, _D)
